# baseline (speedup 1.0000x reference)
import functools

import jax
import jax.numpy as jnp
from jax import lax
from jax.experimental import pallas as pl
from jax.experimental.pallas import tpu as pltpu
from jax.experimental.pallas import tpu_sc as plsc

_CAP = 2048
_PCAP = 1024
_PREC = lax.Precision.HIGHEST


def _sc_info():
    info = plsc.get_sparse_core_info()
    return info.num_cores, info.num_subcores


def _pad_to(x, m):
    pad = (-x.shape[0]) % m
    if pad == 0:
        return x
    return jnp.concatenate([x, jnp.zeros((pad,) + x.shape[1:], x.dtype)])


def _sc_elem_gather(table, idx):
    NC, NS = _sc_info()
    NW = NC * NS
    M = idx.shape[0]
    assert M % (8 * NW) == 0, M
    b = M // NW
    mesh = plsc.VectorSubcoreMesh(core_axis_name="c", subcore_axis_name="s")

    @functools.partial(
        pl.kernel,
        out_type=jax.ShapeDtypeStruct((M,), table.dtype),
        mesh=mesh,
        scratch_types=[
            pltpu.VMEM((b,), jnp.int32),
            pltpu.VMEM((b,), table.dtype),
            pltpu.SemaphoreType.DMA,
        ],
    )
    def k(table_hbm, idx_hbm, out_hbm, idx_v, val_v, sem):
        wid = lax.axis_index("s") * NC + lax.axis_index("c")
        base = wid * b
        pltpu.sync_copy(idx_hbm.at[pl.ds(base, b)], idx_v)
        pltpu.async_copy(table_hbm.at[idx_v], val_v, sem).wait()
        pltpu.sync_copy(val_v, out_hbm.at[pl.ds(base, b)])

    return k(table, idx)


def _sc_tri_columns(f_flat, vorder, vx, vy, vz):
    NC, NS = _sc_info()
    NW = NC * NS
    M = f_flat.shape[0]
    assert M % (8 * NW) == 0, M
    b = M // NW
    mesh = plsc.VectorSubcoreMesh(core_axis_name="c", subcore_axis_name="s")
    cols = jax.ShapeDtypeStruct((M,), vx.dtype)

    @functools.partial(
        pl.kernel,
        out_type=(cols, cols, cols),
        mesh=mesh,
        scratch_types=[
            pltpu.VMEM((b,), jnp.int32),
            pltpu.VMEM((b,), jnp.int32),
            pltpu.VMEM((b,), vx.dtype),
            pltpu.VMEM((b,), vx.dtype),
            pltpu.VMEM((b,), vx.dtype),
            pltpu.SemaphoreType.DMA,
            pltpu.SemaphoreType.DMA,
            pltpu.SemaphoreType.DMA,
            pltpu.SemaphoreType.DMA,
        ],
    )
    def k(fidx_hbm, vord_hbm, vx_hbm, vy_hbm, vz_hbm,
          ox_hbm, oy_hbm, oz_hbm, idx_v, g_v, tx_v, ty_v, tz_v,
          sem, semx, semy, semz):
        wid = lax.axis_index("s") * NC + lax.axis_index("c")
        base = wid * b
        pltpu.sync_copy(fidx_hbm.at[pl.ds(base, b)], idx_v)
        pltpu.async_copy(vord_hbm.at[idx_v], g_v, sem).wait()
        cx = pltpu.async_copy(vx_hbm.at[g_v], tx_v, semx)
        cy = pltpu.async_copy(vy_hbm.at[g_v], ty_v, semy)
        cz = pltpu.async_copy(vz_hbm.at[g_v], tz_v, semz)
        cx.wait()
        pltpu.sync_copy(tx_v, ox_hbm.at[pl.ds(base, b)])
        cy.wait()
        pltpu.sync_copy(ty_v, oy_hbm.at[pl.ds(base, b)])
        cz.wait()
        pltpu.sync_copy(tz_v, oz_hbm.at[pl.ds(base, b)])

    return k(f_flat, vorder, vx, vy, vz)


def _graph_from_columns(c, F):
    eps = 1e-8

    def sub(p, q):
        return [p[l] - q[l] for l in range(3)]

    def dot3(p, q):
        return p[0] * q[0] + p[1] * q[1] + p[2] * q[2]

    def norm3(p):
        return jnp.sqrt(dot3(p, p))

    def unit(p):
        n = norm3(p) + eps
        return [p[l] / n for l in range(3)]

    v0, v1, v2 = c
    e1 = sub(v1, v0)
    e2 = sub(v2, v0)
    nx = e1[1] * e2[2] - e1[2] * e2[1]
    ny = e1[2] * e2[0] - e1[0] * e2[2]
    nz = e1[0] * e2[1] - e1[1] * e2[0]
    nn = jnp.sqrt(nx * nx + ny * ny + nz * nz)
    area = nn * 0.5
    inn = 1.0 / (nn + eps)

    def ang(p, q):
        return jnp.arccos(jnp.clip(dot3(unit(p), unit(q)), -1.0, 1.0))

    a0 = ang(sub(v1, v0), sub(v2, v0))
    a1 = ang(sub(v0, v1), sub(v2, v1))
    a2 = ang(sub(v0, v2), sub(v1, v2))
    colset = [
        v0[0], v0[1], v0[2], v1[0], v1[1], v1[2], v2[0], v2[1], v2[2],
        nx * inn, ny * inn, nz * inn, a0, a1, a2, area,
    ]
    return jnp.stack([col[:F] for col in colset], axis=1)


def _mlp_body(x_ref, w1_ref, b1_ref, w2_ref, b2_ref, o_ref):
    h = jnp.maximum(
        jnp.dot(x_ref[...], w1_ref[...], precision=_PREC) + b1_ref[...], 0.0
    )
    o_ref[...] = jnp.maximum(
        jnp.dot(h, w2_ref[...], precision=_PREC) + b2_ref[...], 0.0
    )


@functools.partial(jax.jit, static_argnames=("block_rows",))
def _mlp(x, W1, b1, W2, b2, block_rows=1024):
    F, Din = x.shape
    Dmid = W1.shape[1]
    Dout = W2.shape[1]
    return pl.pallas_call(
        _mlp_body,
        grid=(pl.cdiv(F, block_rows),),
        in_specs=[
            pl.BlockSpec((block_rows, Din), lambda i: (i, 0)),
            pl.BlockSpec((Din, Dmid), lambda i: (0, 0)),
            pl.BlockSpec((1, Dmid), lambda i: (0, 0)),
            pl.BlockSpec((Dmid, Dout), lambda i: (0, 0)),
            pl.BlockSpec((1, Dout), lambda i: (0, 0)),
        ],
        out_specs=pl.BlockSpec((block_rows, Dout), lambda i: (i, 0)),
        out_shape=jax.ShapeDtypeStruct((F, Dout), jnp.float32),
    )(x, W1, b1.reshape(1, Dmid), W2, b2.reshape(1, Dout))


def _dense_body(x_ref, m_ref, ws_ref, wn_ref, b_ref, o_ref):
    acc = jnp.dot(x_ref[...], ws_ref[...], precision=_PREC)
    acc = acc + jnp.dot(m_ref[...], wn_ref[...], precision=_PREC)
    o_ref[...] = jnp.maximum(acc + b_ref[...], 0.0)


@functools.partial(jax.jit, static_argnames=("block_rows",))
def _dense_layer(x, mean, Ws, Wn, b, block_rows=1024):
    F, Din = x.shape
    Dout = Ws.shape[1]
    return pl.pallas_call(
        _dense_body,
        grid=(pl.cdiv(F, block_rows),),
        in_specs=[
            pl.BlockSpec((block_rows, Din), lambda i: (i, 0)),
            pl.BlockSpec((block_rows, Din), lambda i: (i, 0)),
            pl.BlockSpec((Din, Dout), lambda i: (0, 0)),
            pl.BlockSpec((Din, Dout), lambda i: (0, 0)),
            pl.BlockSpec((1, Dout), lambda i: (0, 0)),
        ],
        out_specs=pl.BlockSpec((block_rows, Dout), lambda i: (i, 0)),
        out_shape=jax.ShapeDtypeStruct((F, Dout), jnp.float32),
    )(x, mean, Ws, Wn, b.reshape(1, Dout))


def _bag(x, N6, sw, cnt):
    F = x.shape[0]
    acc = sw[:, None] * x
    for t in range(6):
        idx = N6[:, t]
        valid = idx < F
        acc = acc + jnp.where(valid[:, None], x[jnp.minimum(idx, F - 1)], 0.0)
    return acc / jnp.maximum(cnt, 1.0)[:, None]


def kernel(vertices, faces, Ws1, Wn1, b1, Ws2, Wn2, b2):
    faces = faces.astype(jnp.int32)
    F = faces.shape[0]
    S = 3 * F
    Nv = vertices.shape[0]

    vorder = jnp.lexsort(
        (vertices[:, 2], vertices[:, 1], vertices[:, 0])
    ).astype(jnp.int32)
    inv = (
        jnp.zeros((Nv,), jnp.int32)
        .at[vorder]
        .set(jnp.arange(Nv, dtype=jnp.int32))
    )
    faces_flat = _pad_to(faces.reshape(-1), 256)
    f = _sc_elem_gather(inv, faces_flat)[:S].reshape(F, 3)
    fa, fb, fc = f[:, 0], f[:, 1], f[:, 2]
    lo01, hi01 = jnp.minimum(fa, fb), jnp.maximum(fa, fb)
    g0 = jnp.minimum(lo01, fc)
    g2 = jnp.maximum(hi01, fc)
    g1 = jnp.minimum(jnp.maximum(lo01, fc), hi01)
    f = jnp.stack([g0, g1, g2], axis=1)
    forder = jnp.lexsort((f[:, 2], g0 * Nv + g1)).astype(jnp.int32)
    finv = (
        jnp.zeros((F,), jnp.int32)
        .at[forder]
        .set(jnp.arange(F, dtype=jnp.int32))
    )

    slot_src = (
        forder[None, :] * 3 + jnp.arange(3, dtype=jnp.int32)[:, None]
    ).reshape(-1)
    f_flat = f.reshape(-1)
    fvals = _sc_elem_gather(_pad_to(f_flat, 256), _pad_to(slot_src, 256))
    cx, cy, cz = _sc_tri_columns(
        fvals, vorder, vertices[:, 0], vertices[:, 1], vertices[:, 2]
    )
    c = [
        [cx[j * F : (j + 1) * F], cy[j * F : (j + 1) * F], cz[j * F : (j + 1) * F]]
        for j in range(3)
    ]
    graph = _graph_from_columns(c, F)

    av = f
    bv = jnp.roll(f, -1, axis=1)
    h = (jnp.minimum(av, bv) * Nv + jnp.maximum(av, bv)).reshape(-1)
    sslot = (finv[:, None] * 3 + jnp.arange(3, dtype=jnp.int32)).reshape(-1)
    hs, order = lax.sort((h, sslot), num_keys=2)
    fs = order // 3

    same_l = jnp.concatenate([jnp.zeros((1,), jnp.bool_), hs[1:] == hs[:-1]])
    npairs = jnp.sum(same_l.astype(jnp.int32))
    f0 = fs[0]

    def fast(ops):
        graph, fs, same_l, f0, Ws1, Wn1, b1, Ws2, Wn2, b2 = ops
        out_d = _mlp(graph, Ws1 + Wn1, b1, Ws2 + Wn2, b2)

        (P,) = jnp.nonzero(same_l, size=_PCAP, fill_value=S)
        valid = P < S
        Pc = jnp.minimum(P, S - 1)
        fs2 = fs[jnp.concatenate([Pc, jnp.maximum(Pc - 1, 0)])]
        rP = fs2[:_PCAP]
        sP = fs2[_PCAP:]
        ends = jnp.concatenate([rP, sP])
        vmask2 = jnp.concatenate([valid, valid])
        ends_s = jnp.where(vmask2, ends, F + 1)

        hasnbr = jnp.zeros((F,), jnp.bool_).at[ends_s].set(True, mode="drop")
        (idxA,) = jnp.nonzero(hasnbr, size=_CAP, fill_value=F + 1)
        idxAc = jnp.minimum(idxA, F - 1)
        posA = jnp.full((F + 2,), _CAP, jnp.int32).at[idxA].set(
            jnp.arange(_CAP, dtype=jnp.int32), mode="drop"
        )
        posA = posA.at[F].set(_CAP).at[F + 1].set(_CAP)

        pos2 = posA[jnp.where(vmask2, ends, F)]
        pr = pos2[:_PCAP]
        ps = pos2[_PCAP:]

        onesv = jnp.where(valid, 1.0, 0.0)
        nLR = (
            jnp.zeros((2 * (_CAP + 1),), jnp.float32)
            .at[jnp.concatenate([pr, ps + (_CAP + 1)])]
            .add(jnp.concatenate([onesv, onesv]))
        )
        nL = nLR[:_CAP]
        nR = nLR[_CAP + 1 : 2 * _CAP + 1]
        pos0A = (idxA == f0).astype(jnp.float32)
        cntA = jnp.maximum(6.0 - nL + nR - 2.0 * pos0A, 1.0)[:, None]
        swA = (6.0 - 2.0 * nL - 2.0 * pos0A)[:, None]

        rev_ends = jnp.concatenate([sP, rP])
        rev_valid = jnp.concatenate([valid, valid])
        rev_src = jnp.where(rev_valid, jnp.minimum(rev_ends, F - 1), 0)
        gidx = jnp.concatenate([idxAc, rev_src])
        gall = graph[gidx]
        xA = gall[:_CAP]
        msg1 = jnp.where(rev_valid[:, None], gall[_CAP:], 0.0)
        rev2 = jnp.concatenate([pos2[_PCAP:], pos2[:_PCAP]])
        nbr1 = (
            jnp.zeros((_CAP + 1, xA.shape[1]), jnp.float32)
            .at[pos2]
            .add(msg1)[:_CAP]
        )
        mean1 = (nbr1 + swA * xA) / cntA
        hA = jax.nn.relu(xA @ Ws1 + mean1 @ Wn1 + b1)
        hA_pad = jnp.concatenate([hA, jnp.zeros((1, hA.shape[1]), hA.dtype)])
        nbr2 = (
            jnp.zeros((_CAP + 1, hA.shape[1]), jnp.float32)
            .at[pos2]
            .add(hA_pad[rev2])[:_CAP]
        )
        mean2 = (nbr2 + swA * hA) / cntA
        outA = jax.nn.relu(hA @ Ws2 + mean2 @ Wn2 + b2)
        posAf = jnp.minimum(posA[:F], _CAP - 1)
        patched = outA[posAf]
        return jnp.where(hasnbr[:, None], patched, out_d)

    def slow(ops):
        graph, fs, same_l, f0, Ws1, Wn1, b1, Ws2, Wn2, b2 = ops
        same_r = jnp.concatenate([same_l[1:], jnp.zeros((1,), jnp.bool_)])
        pos = jnp.arange(S)
        selfw = jnp.where(jnp.logical_and(pos > 0, ~same_l), 2.0, 0.0)
        percnt = same_l.astype(jnp.float32) + same_r.astype(jnp.float32) + selfw
        left_nb = jnp.where(same_l, jnp.roll(fs, 1), F)
        right_nb = jnp.where(same_r, jnp.roll(fs, -1), F)

        def to_slot(v):
            return jnp.zeros(S, v.dtype).at[order].set(v)

        LN = to_slot(left_nb).reshape(F, 3)
        RN = to_slot(right_nb).reshape(F, 3)
        sw = to_slot(selfw).reshape(F, 3).sum(axis=1)
        cnt = to_slot(percnt).reshape(F, 3).sum(axis=1)
        N6 = jnp.concatenate([LN, RN], axis=1).astype(jnp.int32)

        mean1 = _bag(graph, N6, sw, cnt)
        hh = _dense_layer(graph, mean1, Ws1, Wn1, b1)
        mean2 = _bag(hh, N6, sw, cnt)
        return _dense_layer(hh, mean2, Ws2, Wn2, b2)

    ops = (graph, fs, same_l, f0, Ws1, Wn1, b1, Ws2, Wn2, b2)
    return lax.cond(npairs <= min(_PCAP, _CAP // 2), fast, slow, ops)


# speedup vs baseline: 1.0740x; 1.0740x over previous
import functools

import jax
import jax.numpy as jnp
from jax import lax
from jax.experimental import pallas as pl
from jax.experimental.pallas import tpu as pltpu
from jax.experimental.pallas import tpu_sc as plsc

_CAP = 2048
_PCAP = 1024
_PREC = lax.Precision.HIGHEST


def _sc_info():
    info = plsc.get_sparse_core_info()
    return info.num_cores, info.num_subcores


def _pad_to(x, m):
    pad = (-x.shape[0]) % m
    if pad == 0:
        return x
    return jnp.concatenate([x, jnp.zeros((pad,) + x.shape[1:], x.dtype)])


def _sc_elem_gather(table, idx):
    NC, NS = _sc_info()
    NW = NC * NS
    M = idx.shape[0]
    assert M % (8 * NW) == 0, M
    b = M // NW
    mesh = plsc.VectorSubcoreMesh(core_axis_name="c", subcore_axis_name="s")

    @functools.partial(
        pl.kernel,
        out_type=jax.ShapeDtypeStruct((M,), table.dtype),
        mesh=mesh,
        scratch_types=[
            pltpu.VMEM((b,), jnp.int32),
            pltpu.VMEM((b,), table.dtype),
            pltpu.SemaphoreType.DMA,
        ],
    )
    def k(table_hbm, idx_hbm, out_hbm, idx_v, val_v, sem):
        wid = lax.axis_index("s") * NC + lax.axis_index("c")
        base = wid * b
        pltpu.sync_copy(idx_hbm.at[pl.ds(base, b)], idx_v)
        pltpu.async_copy(table_hbm.at[idx_v], val_v, sem).wait()
        pltpu.sync_copy(val_v, out_hbm.at[pl.ds(base, b)])

    return k(table, idx)


def _sc_tri_columns(f_flat, vorder, vx, vy, vz):
    NC, NS = _sc_info()
    NW = NC * NS
    M = f_flat.shape[0]
    assert M % (8 * NW) == 0, M
    b = M // NW
    mesh = plsc.VectorSubcoreMesh(core_axis_name="c", subcore_axis_name="s")
    cols = jax.ShapeDtypeStruct((M,), vx.dtype)

    @functools.partial(
        pl.kernel,
        out_type=(cols, cols, cols),
        mesh=mesh,
        scratch_types=[
            pltpu.VMEM((b,), jnp.int32),
            pltpu.VMEM((b,), jnp.int32),
            pltpu.VMEM((b,), vx.dtype),
            pltpu.VMEM((b,), vx.dtype),
            pltpu.VMEM((b,), vx.dtype),
            pltpu.SemaphoreType.DMA,
            pltpu.SemaphoreType.DMA,
            pltpu.SemaphoreType.DMA,
            pltpu.SemaphoreType.DMA,
        ],
    )
    def k(fidx_hbm, vord_hbm, vx_hbm, vy_hbm, vz_hbm,
          ox_hbm, oy_hbm, oz_hbm, idx_v, g_v, tx_v, ty_v, tz_v,
          sem, semx, semy, semz):
        wid = lax.axis_index("s") * NC + lax.axis_index("c")
        base = wid * b
        pltpu.sync_copy(fidx_hbm.at[pl.ds(base, b)], idx_v)
        pltpu.async_copy(vord_hbm.at[idx_v], g_v, sem).wait()
        cx = pltpu.async_copy(vx_hbm.at[g_v], tx_v, semx)
        cy = pltpu.async_copy(vy_hbm.at[g_v], ty_v, semy)
        cz = pltpu.async_copy(vz_hbm.at[g_v], tz_v, semz)
        cx.wait()
        pltpu.sync_copy(tx_v, ox_hbm.at[pl.ds(base, b)])
        cy.wait()
        pltpu.sync_copy(ty_v, oy_hbm.at[pl.ds(base, b)])
        cz.wait()
        pltpu.sync_copy(tz_v, oz_hbm.at[pl.ds(base, b)])

    return k(f_flat, vorder, vx, vy, vz)


def _graph_from_columns(c, F):
    eps = 1e-8

    def sub(p, q):
        return [p[l] - q[l] for l in range(3)]

    def dot3(p, q):
        return p[0] * q[0] + p[1] * q[1] + p[2] * q[2]

    def norm3(p):
        return jnp.sqrt(dot3(p, p))

    def unit(p):
        n = norm3(p) + eps
        return [p[l] / n for l in range(3)]

    v0, v1, v2 = c
    e1 = sub(v1, v0)
    e2 = sub(v2, v0)
    nx = e1[1] * e2[2] - e1[2] * e2[1]
    ny = e1[2] * e2[0] - e1[0] * e2[2]
    nz = e1[0] * e2[1] - e1[1] * e2[0]
    nn = jnp.sqrt(nx * nx + ny * ny + nz * nz)
    area = nn * 0.5
    inn = 1.0 / (nn + eps)

    def ang(p, q):
        return jnp.arccos(jnp.clip(dot3(unit(p), unit(q)), -1.0, 1.0))

    a0 = ang(sub(v1, v0), sub(v2, v0))
    a1 = ang(sub(v0, v1), sub(v2, v1))
    a2 = ang(sub(v0, v2), sub(v1, v2))
    colset = [
        v0[0], v0[1], v0[2], v1[0], v1[1], v1[2], v2[0], v2[1], v2[2],
        nx * inn, ny * inn, nz * inn, a0, a1, a2, area,
    ]
    return jnp.stack([col[:F] for col in colset], axis=1)


def _mlp_body(x_ref, w1_ref, b1_ref, w2_ref, b2_ref, o_ref):
    h = jnp.maximum(
        jnp.dot(x_ref[...], w1_ref[...], precision=_PREC) + b1_ref[...], 0.0
    )
    o_ref[...] = jnp.maximum(
        jnp.dot(h, w2_ref[...], precision=_PREC) + b2_ref[...], 0.0
    )


@functools.partial(jax.jit, static_argnames=("block_rows",))
def _mlp(x, W1, b1, W2, b2, block_rows=1024):
    F, Din = x.shape
    Dmid = W1.shape[1]
    Dout = W2.shape[1]
    return pl.pallas_call(
        _mlp_body,
        grid=(pl.cdiv(F, block_rows),),
        in_specs=[
            pl.BlockSpec((block_rows, Din), lambda i: (i, 0)),
            pl.BlockSpec((Din, Dmid), lambda i: (0, 0)),
            pl.BlockSpec((1, Dmid), lambda i: (0, 0)),
            pl.BlockSpec((Dmid, Dout), lambda i: (0, 0)),
            pl.BlockSpec((1, Dout), lambda i: (0, 0)),
        ],
        out_specs=pl.BlockSpec((block_rows, Dout), lambda i: (i, 0)),
        out_shape=jax.ShapeDtypeStruct((F, Dout), jnp.float32),
    )(x, W1, b1.reshape(1, Dmid), W2, b2.reshape(1, Dout))


def _dense_body(x_ref, m_ref, ws_ref, wn_ref, b_ref, o_ref):
    acc = jnp.dot(x_ref[...], ws_ref[...], precision=_PREC)
    acc = acc + jnp.dot(m_ref[...], wn_ref[...], precision=_PREC)
    o_ref[...] = jnp.maximum(acc + b_ref[...], 0.0)


@functools.partial(jax.jit, static_argnames=("block_rows",))
def _dense_layer(x, mean, Ws, Wn, b, block_rows=1024):
    F, Din = x.shape
    Dout = Ws.shape[1]
    return pl.pallas_call(
        _dense_body,
        grid=(pl.cdiv(F, block_rows),),
        in_specs=[
            pl.BlockSpec((block_rows, Din), lambda i: (i, 0)),
            pl.BlockSpec((block_rows, Din), lambda i: (i, 0)),
            pl.BlockSpec((Din, Dout), lambda i: (0, 0)),
            pl.BlockSpec((Din, Dout), lambda i: (0, 0)),
            pl.BlockSpec((1, Dout), lambda i: (0, 0)),
        ],
        out_specs=pl.BlockSpec((block_rows, Dout), lambda i: (i, 0)),
        out_shape=jax.ShapeDtypeStruct((F, Dout), jnp.float32),
    )(x, mean, Ws, Wn, b.reshape(1, Dout))


def _bag(x, N6, sw, cnt):
    F = x.shape[0]
    acc = sw[:, None] * x
    for t in range(6):
        idx = N6[:, t]
        valid = idx < F
        acc = acc + jnp.where(valid[:, None], x[jnp.minimum(idx, F - 1)], 0.0)
    return acc / jnp.maximum(cnt, 1.0)[:, None]


def kernel(vertices, faces, Ws1, Wn1, b1, Ws2, Wn2, b2):
    faces = faces.astype(jnp.int32)
    F = faces.shape[0]
    S = 3 * F
    Nv = vertices.shape[0]

    vorder = jnp.lexsort(
        (vertices[:, 2], vertices[:, 1], vertices[:, 0])
    ).astype(jnp.int32)
    inv = (
        jnp.zeros((Nv,), jnp.int32)
        .at[vorder]
        .set(jnp.arange(Nv, dtype=jnp.int32))
    )
    faces_flat = _pad_to(faces.reshape(-1), 256)
    f = _sc_elem_gather(inv, faces_flat)[:S].reshape(F, 3)
    fa, fb, fc = f[:, 0], f[:, 1], f[:, 2]
    lo01, hi01 = jnp.minimum(fa, fb), jnp.maximum(fa, fb)
    g0 = jnp.minimum(lo01, fc)
    g2 = jnp.maximum(hi01, fc)
    g1 = jnp.minimum(jnp.maximum(lo01, fc), hi01)
    f = jnp.stack([g0, g1, g2], axis=1)
    forder = jnp.lexsort((f[:, 2], g0 * Nv + g1)).astype(jnp.int32)
    finv = (
        jnp.zeros((F,), jnp.int32)
        .at[forder]
        .set(jnp.arange(F, dtype=jnp.int32))
    )

    slot_src = (
        forder[None, :] * 3 + jnp.arange(3, dtype=jnp.int32)[:, None]
    ).reshape(-1)
    f_flat = f.reshape(-1)
    fvals = _sc_elem_gather(_pad_to(f_flat, 256), _pad_to(slot_src, 256))
    cx, cy, cz = _sc_tri_columns(
        fvals, vorder, vertices[:, 0], vertices[:, 1], vertices[:, 2]
    )
    c = [
        [cx[j * F : (j + 1) * F], cy[j * F : (j + 1) * F], cz[j * F : (j + 1) * F]]
        for j in range(3)
    ]
    graph = _graph_from_columns(c, F)

    av = f
    bv = jnp.roll(f, -1, axis=1)
    h = (jnp.minimum(av, bv) * Nv + jnp.maximum(av, bv)).reshape(-1)
    sslot = (finv[:, None] * 3 + jnp.arange(3, dtype=jnp.int32)).reshape(-1)
    hs, order = lax.sort((h, sslot), num_keys=2)
    fs = order // 3

    same_l = jnp.concatenate([jnp.zeros((1,), jnp.bool_), hs[1:] == hs[:-1]])
    npairs = jnp.sum(same_l.astype(jnp.int32))
    f0 = fs[0]

    def fast(ops):
        graph, fs, same_l, f0, Ws1, Wn1, b1, Ws2, Wn2, b2 = ops
        out_d = _mlp(graph, Ws1 + Wn1, b1, Ws2 + Wn2, b2)

        (P,) = jnp.nonzero(same_l, size=_PCAP, fill_value=S)
        valid = P < S
        Pc = jnp.minimum(P, S - 1)
        fs2 = fs[jnp.concatenate([Pc, jnp.maximum(Pc - 1, 0)])]
        rP = fs2[:_PCAP]
        sP = fs2[_PCAP:]
        ends = jnp.concatenate([rP, sP])
        vmask2 = jnp.concatenate([valid, valid])
        ends_s = jnp.where(vmask2, ends, F + 1)

        hasnbr = jnp.zeros((F,), jnp.bool_).at[ends_s].set(True, mode="drop")
        (idxA,) = jnp.nonzero(hasnbr, size=_CAP, fill_value=F + 1)
        idxAc = jnp.minimum(idxA, F - 1)
        posA = jnp.full((F + 2,), _CAP, jnp.int32).at[idxA].set(
            jnp.arange(_CAP, dtype=jnp.int32), mode="drop"
        )
        posA = posA.at[F].set(_CAP).at[F + 1].set(_CAP)

        pos2 = posA[jnp.where(vmask2, ends, F)]
        pr = pos2[:_PCAP]
        ps = pos2[_PCAP:]

        onesv = jnp.where(valid, 1.0, 0.0)
        nLR = (
            jnp.zeros((2 * (_CAP + 1),), jnp.float32)
            .at[jnp.concatenate([pr, ps + (_CAP + 1)])]
            .add(jnp.concatenate([onesv, onesv]))
        )
        nL = nLR[:_CAP]
        nR = nLR[_CAP + 1 : 2 * _CAP + 1]
        pos0A = (idxA == f0).astype(jnp.float32)
        cntA = jnp.maximum(6.0 - nL + nR - 2.0 * pos0A, 1.0)[:, None]
        swA = (6.0 - 2.0 * nL - 2.0 * pos0A)[:, None]

        rev_ends = jnp.concatenate([sP, rP])
        rev_valid = jnp.concatenate([valid, valid])
        rev_src = jnp.where(rev_valid, jnp.minimum(rev_ends, F - 1), 0)
        gidx = jnp.concatenate([idxAc, rev_src])
        gall = graph[gidx]
        xA = gall[:_CAP]
        msg1 = jnp.where(rev_valid[:, None], gall[_CAP:], 0.0)
        rev2 = jnp.concatenate([pos2[_PCAP:], pos2[:_PCAP]])
        nbr1 = (
            jnp.zeros((_CAP + 1, xA.shape[1]), jnp.float32)
            .at[pos2]
            .add(msg1)[:_CAP]
        )
        mean1 = (nbr1 + swA * xA) / cntA
        hA = jax.nn.relu(xA @ Ws1 + mean1 @ Wn1 + b1)
        hA_pad = jnp.concatenate([hA, jnp.zeros((1, hA.shape[1]), hA.dtype)])
        nbr2 = (
            jnp.zeros((_CAP + 1, hA.shape[1]), jnp.float32)
            .at[pos2]
            .add(hA_pad[rev2])[:_CAP]
        )
        mean2 = (nbr2 + swA * hA) / cntA
        outA = jax.nn.relu(hA @ Ws2 + mean2 @ Wn2 + b2)
        return out_d.at[idxA].set(outA, mode="drop")

    def slow(ops):
        graph, fs, same_l, f0, Ws1, Wn1, b1, Ws2, Wn2, b2 = ops
        same_r = jnp.concatenate([same_l[1:], jnp.zeros((1,), jnp.bool_)])
        pos = jnp.arange(S)
        selfw = jnp.where(jnp.logical_and(pos > 0, ~same_l), 2.0, 0.0)
        percnt = same_l.astype(jnp.float32) + same_r.astype(jnp.float32) + selfw
        left_nb = jnp.where(same_l, jnp.roll(fs, 1), F)
        right_nb = jnp.where(same_r, jnp.roll(fs, -1), F)

        def to_slot(v):
            return jnp.zeros(S, v.dtype).at[order].set(v)

        LN = to_slot(left_nb).reshape(F, 3)
        RN = to_slot(right_nb).reshape(F, 3)
        sw = to_slot(selfw).reshape(F, 3).sum(axis=1)
        cnt = to_slot(percnt).reshape(F, 3).sum(axis=1)
        N6 = jnp.concatenate([LN, RN], axis=1).astype(jnp.int32)

        mean1 = _bag(graph, N6, sw, cnt)
        hh = _dense_layer(graph, mean1, Ws1, Wn1, b1)
        mean2 = _bag(hh, N6, sw, cnt)
        return _dense_layer(hh, mean2, Ws2, Wn2, b2)

    ops = (graph, fs, same_l, f0, Ws1, Wn1, b1, Ws2, Wn2, b2)
    return lax.cond(npairs <= min(_PCAP, _CAP // 2), fast, slow, ops)


# speedup vs baseline: 1.1960x; 1.1136x over previous
import functools

import jax
import jax.numpy as jnp
from jax import lax
from jax.experimental import pallas as pl
from jax.experimental.pallas import tpu as pltpu
from jax.experimental.pallas import tpu_sc as plsc

_CAP = 2048
_PCAP = 1024
_PREC = lax.Precision.HIGHEST


def _sc_info():
    info = plsc.get_sparse_core_info()
    return info.num_cores, info.num_subcores


def _pad_to(x, m):
    pad = (-x.shape[0]) % m
    if pad == 0:
        return x
    return jnp.concatenate([x, jnp.zeros((pad,) + x.shape[1:], x.dtype)])


def _sc_elem_gather(table, idx):
    NC, NS = _sc_info()
    NW = NC * NS
    M = idx.shape[0]
    assert M % (8 * NW) == 0, M
    b = M // NW
    mesh = plsc.VectorSubcoreMesh(core_axis_name="c", subcore_axis_name="s")

    @functools.partial(
        pl.kernel,
        out_type=jax.ShapeDtypeStruct((M,), table.dtype),
        mesh=mesh,
        scratch_types=[
            pltpu.VMEM((b,), jnp.int32),
            pltpu.VMEM((b,), table.dtype),
            pltpu.SemaphoreType.DMA,
        ],
    )
    def k(table_hbm, idx_hbm, out_hbm, idx_v, val_v, sem):
        wid = lax.axis_index("s") * NC + lax.axis_index("c")
        base = wid * b
        pltpu.sync_copy(idx_hbm.at[pl.ds(base, b)], idx_v)
        pltpu.async_copy(table_hbm.at[idx_v], val_v, sem).wait()
        pltpu.sync_copy(val_v, out_hbm.at[pl.ds(base, b)])

    return k(table, idx)


def _sc_tri_columns(f_flat, vorder, vx, vy, vz):
    NC, NS = _sc_info()
    NW = NC * NS
    M = f_flat.shape[0]
    assert M % (8 * NW) == 0, M
    b = M // NW
    mesh = plsc.VectorSubcoreMesh(core_axis_name="c", subcore_axis_name="s")
    cols = jax.ShapeDtypeStruct((M,), vx.dtype)

    @functools.partial(
        pl.kernel,
        out_type=(cols, cols, cols),
        mesh=mesh,
        scratch_types=[
            pltpu.VMEM((b,), jnp.int32),
            pltpu.VMEM((b,), jnp.int32),
            pltpu.VMEM((b,), vx.dtype),
            pltpu.VMEM((b,), vx.dtype),
            pltpu.VMEM((b,), vx.dtype),
            pltpu.SemaphoreType.DMA,
            pltpu.SemaphoreType.DMA,
            pltpu.SemaphoreType.DMA,
            pltpu.SemaphoreType.DMA,
        ],
    )
    def k(fidx_hbm, vord_hbm, vx_hbm, vy_hbm, vz_hbm,
          ox_hbm, oy_hbm, oz_hbm, idx_v, g_v, tx_v, ty_v, tz_v,
          sem, semx, semy, semz):
        wid = lax.axis_index("s") * NC + lax.axis_index("c")
        base = wid * b
        pltpu.sync_copy(fidx_hbm.at[pl.ds(base, b)], idx_v)
        pltpu.async_copy(vord_hbm.at[idx_v], g_v, sem).wait()
        cx = pltpu.async_copy(vx_hbm.at[g_v], tx_v, semx)
        cy = pltpu.async_copy(vy_hbm.at[g_v], ty_v, semy)
        cz = pltpu.async_copy(vz_hbm.at[g_v], tz_v, semz)
        cx.wait()
        pltpu.sync_copy(tx_v, ox_hbm.at[pl.ds(base, b)])
        cy.wait()
        pltpu.sync_copy(ty_v, oy_hbm.at[pl.ds(base, b)])
        cz.wait()
        pltpu.sync_copy(tz_v, oz_hbm.at[pl.ds(base, b)])

    return k(f_flat, vorder, vx, vy, vz)


def _graph_from_columns(c, F):
    eps = 1e-8

    def sub(p, q):
        return [p[l] - q[l] for l in range(3)]

    def dot3(p, q):
        return p[0] * q[0] + p[1] * q[1] + p[2] * q[2]

    def norm3(p):
        return jnp.sqrt(dot3(p, p))

    def unit(p):
        n = norm3(p) + eps
        return [p[l] / n for l in range(3)]

    v0, v1, v2 = c
    e1 = sub(v1, v0)
    e2 = sub(v2, v0)
    nx = e1[1] * e2[2] - e1[2] * e2[1]
    ny = e1[2] * e2[0] - e1[0] * e2[2]
    nz = e1[0] * e2[1] - e1[1] * e2[0]
    nn = jnp.sqrt(nx * nx + ny * ny + nz * nz)
    area = nn * 0.5
    inn = 1.0 / (nn + eps)

    def ang(p, q):
        return jnp.arccos(jnp.clip(dot3(unit(p), unit(q)), -1.0, 1.0))

    a0 = ang(sub(v1, v0), sub(v2, v0))
    a1 = ang(sub(v0, v1), sub(v2, v1))
    a2 = ang(sub(v0, v2), sub(v1, v2))
    colset = [
        v0[0], v0[1], v0[2], v1[0], v1[1], v1[2], v2[0], v2[1], v2[2],
        nx * inn, ny * inn, nz * inn, a0, a1, a2, area,
    ]
    return jnp.stack([col[:F] for col in colset], axis=1)


def _mlp_body(x_ref, w1_ref, b1_ref, w2_ref, b2_ref, p_ref, m_ref, o_ref):
    h = jnp.maximum(
        jnp.dot(x_ref[...], w1_ref[...], precision=_PREC) + b1_ref[...], 0.0
    )
    o = jnp.maximum(
        jnp.dot(h, w2_ref[...], precision=_PREC) + b2_ref[...], 0.0
    )
    o_ref[...] = jnp.where(m_ref[...] > 0.0, p_ref[...], o)


@functools.partial(jax.jit, static_argnames=("block_rows",))
def _mlp(x, W1, b1, W2, b2, patch, mask, block_rows=1024):
    F, Din = x.shape
    Dmid = W1.shape[1]
    Dout = W2.shape[1]
    return pl.pallas_call(
        _mlp_body,
        grid=(pl.cdiv(F, block_rows),),
        in_specs=[
            pl.BlockSpec((block_rows, Din), lambda i: (i, 0)),
            pl.BlockSpec((Din, Dmid), lambda i: (0, 0)),
            pl.BlockSpec((1, Dmid), lambda i: (0, 0)),
            pl.BlockSpec((Dmid, Dout), lambda i: (0, 0)),
            pl.BlockSpec((1, Dout), lambda i: (0, 0)),
            pl.BlockSpec((block_rows, Dout), lambda i: (i, 0)),
            pl.BlockSpec((block_rows, 1), lambda i: (i, 0)),
        ],
        out_specs=pl.BlockSpec((block_rows, Dout), lambda i: (i, 0)),
        out_shape=jax.ShapeDtypeStruct((F, Dout), jnp.float32),
    )(x, W1, b1.reshape(1, Dmid), W2, b2.reshape(1, Dout), patch,
      mask.reshape(F, 1))


def _dense_body(x_ref, m_ref, ws_ref, wn_ref, b_ref, o_ref):
    acc = jnp.dot(x_ref[...], ws_ref[...], precision=_PREC)
    acc = acc + jnp.dot(m_ref[...], wn_ref[...], precision=_PREC)
    o_ref[...] = jnp.maximum(acc + b_ref[...], 0.0)


@functools.partial(jax.jit, static_argnames=("block_rows",))
def _dense_layer(x, mean, Ws, Wn, b, block_rows=1024):
    F, Din = x.shape
    Dout = Ws.shape[1]
    return pl.pallas_call(
        _dense_body,
        grid=(pl.cdiv(F, block_rows),),
        in_specs=[
            pl.BlockSpec((block_rows, Din), lambda i: (i, 0)),
            pl.BlockSpec((block_rows, Din), lambda i: (i, 0)),
            pl.BlockSpec((Din, Dout), lambda i: (0, 0)),
            pl.BlockSpec((Din, Dout), lambda i: (0, 0)),
            pl.BlockSpec((1, Dout), lambda i: (0, 0)),
        ],
        out_specs=pl.BlockSpec((block_rows, Dout), lambda i: (i, 0)),
        out_shape=jax.ShapeDtypeStruct((F, Dout), jnp.float32),
    )(x, mean, Ws, Wn, b.reshape(1, Dout))


def _bag(x, N6, sw, cnt):
    F = x.shape[0]
    acc = sw[:, None] * x
    for t in range(6):
        idx = N6[:, t]
        valid = idx < F
        acc = acc + jnp.where(valid[:, None], x[jnp.minimum(idx, F - 1)], 0.0)
    return acc / jnp.maximum(cnt, 1.0)[:, None]


def kernel(vertices, faces, Ws1, Wn1, b1, Ws2, Wn2, b2):
    faces = faces.astype(jnp.int32)
    F = faces.shape[0]
    S = 3 * F
    Nv = vertices.shape[0]

    vorder = jnp.lexsort(
        (vertices[:, 2], vertices[:, 1], vertices[:, 0])
    ).astype(jnp.int32)
    inv = (
        jnp.zeros((Nv,), jnp.int32)
        .at[vorder]
        .set(jnp.arange(Nv, dtype=jnp.int32))
    )
    faces_flat = _pad_to(faces.reshape(-1), 256)
    f = _sc_elem_gather(inv, faces_flat)[:S].reshape(F, 3)
    fa, fb, fc = f[:, 0], f[:, 1], f[:, 2]
    lo01, hi01 = jnp.minimum(fa, fb), jnp.maximum(fa, fb)
    g0 = jnp.minimum(lo01, fc)
    g2 = jnp.maximum(hi01, fc)
    g1 = jnp.minimum(jnp.maximum(lo01, fc), hi01)
    f = jnp.stack([g0, g1, g2], axis=1)
    forder = jnp.lexsort((f[:, 2], g0 * Nv + g1)).astype(jnp.int32)
    finv = (
        jnp.zeros((F,), jnp.int32)
        .at[forder]
        .set(jnp.arange(F, dtype=jnp.int32))
    )

    slot_src = (
        forder[None, :] * 3 + jnp.arange(3, dtype=jnp.int32)[:, None]
    ).reshape(-1)
    f_flat = f.reshape(-1)
    fvals = _sc_elem_gather(_pad_to(f_flat, 256), _pad_to(slot_src, 256))
    cx, cy, cz = _sc_tri_columns(
        fvals, vorder, vertices[:, 0], vertices[:, 1], vertices[:, 2]
    )
    c = [
        [cx[j * F : (j + 1) * F], cy[j * F : (j + 1) * F], cz[j * F : (j + 1) * F]]
        for j in range(3)
    ]
    graph = _graph_from_columns(c, F)

    av = f
    bv = jnp.roll(f, -1, axis=1)
    h = (jnp.minimum(av, bv) * Nv + jnp.maximum(av, bv)).reshape(-1)
    sslot = (finv[:, None] * 3 + jnp.arange(3, dtype=jnp.int32)).reshape(-1)
    hs, order = lax.sort((h, sslot), num_keys=2)
    fs = order // 3

    same_l = jnp.concatenate([jnp.zeros((1,), jnp.bool_), hs[1:] == hs[:-1]])
    npairs = jnp.sum(same_l.astype(jnp.int32))
    f0 = fs[0]

    def fast(ops):
        graph, fs, same_l, f0, Ws1, Wn1, b1, Ws2, Wn2, b2 = ops

        (P,) = jnp.nonzero(same_l, size=_PCAP, fill_value=S)
        valid = P < S
        Pc = jnp.minimum(P, S - 1)
        idx2 = jnp.concatenate([Pc, jnp.maximum(Pc - 1, 0)])
        fs2 = _sc_elem_gather(fs, idx2)
        rP = fs2[:_PCAP]
        sP = fs2[_PCAP:]
        ends = fs2
        vmask2 = jnp.concatenate([valid, valid])
        ends_s = jnp.where(vmask2, ends, F + 1)

        hasnbr = jnp.zeros((F,), jnp.bool_).at[ends_s].set(True, mode="drop")
        (idxA,) = jnp.nonzero(hasnbr, size=_CAP, fill_value=F + 1)
        idxAc = jnp.minimum(idxA, F - 1)
        posA = jnp.full((F + 2,), _CAP, jnp.int32).at[idxA].set(
            jnp.arange(_CAP, dtype=jnp.int32), mode="drop"
        )
        posA = posA.at[F].set(_CAP).at[F + 1].set(_CAP)

        pos2 = _sc_elem_gather(posA, jnp.where(vmask2, ends, F))
        pr = pos2[:_PCAP]
        ps = pos2[_PCAP:]
        rev2 = jnp.concatenate([ps, pr])

        onesv2 = jnp.where(vmask2, 1.0, 0.0)
        adj_flat = (
            jnp.zeros(((_CAP + 1) * (_CAP + 1),), jnp.float32)
            .at[pos2 * (_CAP + 1) + rev2]
            .add(onesv2)
        )
        Adj = adj_flat.reshape(_CAP + 1, _CAP + 1)[:_CAP, :_CAP]

        nL = jnp.zeros((_CAP + 1,), jnp.float32).at[pr].add(
            jnp.where(valid, 1.0, 0.0)
        )[:_CAP]
        nR = jnp.zeros((_CAP + 1,), jnp.float32).at[ps].add(
            jnp.where(valid, 1.0, 0.0)
        )[:_CAP]
        pos0A = (idxA == f0).astype(jnp.float32)
        cntA = jnp.maximum(6.0 - nL + nR - 2.0 * pos0A, 1.0)[:, None]
        swA = (6.0 - 2.0 * nL - 2.0 * pos0A)[:, None]

        xA = graph[idxAc]
        nbr1 = jnp.dot(Adj, xA, precision=_PREC)
        mean1 = (nbr1 + swA * xA) / cntA
        hA = jax.nn.relu(xA @ Ws1 + mean1 @ Wn1 + b1)
        nbr2 = jnp.dot(Adj, hA, precision=_PREC)
        mean2 = (nbr2 + swA * hA) / cntA
        outA = jax.nn.relu(hA @ Ws2 + mean2 @ Wn2 + b2)

        patch = jnp.zeros((F, outA.shape[1]), jnp.float32).at[idxA].set(
            outA, mode="drop"
        )
        return _mlp(
            graph, Ws1 + Wn1, b1, Ws2 + Wn2, b2, patch,
            hasnbr.astype(jnp.float32),
        )

    def slow(ops):
        graph, fs, same_l, f0, Ws1, Wn1, b1, Ws2, Wn2, b2 = ops
        same_r = jnp.concatenate([same_l[1:], jnp.zeros((1,), jnp.bool_)])
        pos = jnp.arange(S)
        selfw = jnp.where(jnp.logical_and(pos > 0, ~same_l), 2.0, 0.0)
        percnt = same_l.astype(jnp.float32) + same_r.astype(jnp.float32) + selfw
        left_nb = jnp.where(same_l, jnp.roll(fs, 1), F)
        right_nb = jnp.where(same_r, jnp.roll(fs, -1), F)

        def to_slot(v):
            return jnp.zeros(S, v.dtype).at[order].set(v)

        LN = to_slot(left_nb).reshape(F, 3)
        RN = to_slot(right_nb).reshape(F, 3)
        sw = to_slot(selfw).reshape(F, 3).sum(axis=1)
        cnt = to_slot(percnt).reshape(F, 3).sum(axis=1)
        N6 = jnp.concatenate([LN, RN], axis=1).astype(jnp.int32)

        mean1 = _bag(graph, N6, sw, cnt)
        hh = _dense_layer(graph, mean1, Ws1, Wn1, b1)
        mean2 = _bag(hh, N6, sw, cnt)
        return _dense_layer(hh, mean2, Ws2, Wn2, b2)

    ops = (graph, fs, same_l, f0, Ws1, Wn1, b1, Ws2, Wn2, b2)
    return lax.cond(npairs <= min(_PCAP, _CAP // 2), fast, slow, ops)


# speedup vs baseline: 1.2001x; 1.0034x over previous
import functools

import jax
import jax.numpy as jnp
from jax import lax
from jax.experimental import pallas as pl
from jax.experimental.pallas import tpu as pltpu
from jax.experimental.pallas import tpu_sc as plsc

_CAP = 2048
_PCAP = 1024
_PREC = lax.Precision.HIGHEST


def _sc_info():
    info = plsc.get_sparse_core_info()
    return info.num_cores, info.num_subcores


def _pad_to(x, m):
    pad = (-x.shape[0]) % m
    if pad == 0:
        return x
    return jnp.concatenate([x, jnp.zeros((pad,) + x.shape[1:], x.dtype)])


def _sc_elem_gather(table, idx):
    NC, NS = _sc_info()
    NW = NC * NS
    M = idx.shape[0]
    assert M % (8 * NW) == 0, M
    b = M // NW
    mesh = plsc.VectorSubcoreMesh(core_axis_name="c", subcore_axis_name="s")

    @functools.partial(
        pl.kernel,
        out_type=jax.ShapeDtypeStruct((M,), table.dtype),
        mesh=mesh,
        scratch_types=[
            pltpu.VMEM((b,), jnp.int32),
            pltpu.VMEM((b,), table.dtype),
            pltpu.SemaphoreType.DMA,
        ],
    )
    def k(table_hbm, idx_hbm, out_hbm, idx_v, val_v, sem):
        wid = lax.axis_index("s") * NC + lax.axis_index("c")
        base = wid * b
        pltpu.sync_copy(idx_hbm.at[pl.ds(base, b)], idx_v)
        pltpu.async_copy(table_hbm.at[idx_v], val_v, sem).wait()
        pltpu.sync_copy(val_v, out_hbm.at[pl.ds(base, b)])

    return k(table, idx)


def _sc_tri_columns(f_flat, vorder, vx, vy, vz):
    NC, NS = _sc_info()
    NW = NC * NS
    M = f_flat.shape[0]
    assert M % (8 * NW) == 0, M
    b = M // NW
    mesh = plsc.VectorSubcoreMesh(core_axis_name="c", subcore_axis_name="s")
    cols = jax.ShapeDtypeStruct((M,), vx.dtype)

    @functools.partial(
        pl.kernel,
        out_type=(cols, cols, cols),
        mesh=mesh,
        scratch_types=[
            pltpu.VMEM((b,), jnp.int32),
            pltpu.VMEM((b,), jnp.int32),
            pltpu.VMEM((b,), vx.dtype),
            pltpu.VMEM((b,), vx.dtype),
            pltpu.VMEM((b,), vx.dtype),
            pltpu.SemaphoreType.DMA,
            pltpu.SemaphoreType.DMA,
            pltpu.SemaphoreType.DMA,
            pltpu.SemaphoreType.DMA,
        ],
    )
    def k(fidx_hbm, vord_hbm, vx_hbm, vy_hbm, vz_hbm,
          ox_hbm, oy_hbm, oz_hbm, idx_v, g_v, tx_v, ty_v, tz_v,
          sem, semx, semy, semz):
        wid = lax.axis_index("s") * NC + lax.axis_index("c")
        base = wid * b
        pltpu.sync_copy(fidx_hbm.at[pl.ds(base, b)], idx_v)
        pltpu.async_copy(vord_hbm.at[idx_v], g_v, sem).wait()
        cx = pltpu.async_copy(vx_hbm.at[g_v], tx_v, semx)
        cy = pltpu.async_copy(vy_hbm.at[g_v], ty_v, semy)
        cz = pltpu.async_copy(vz_hbm.at[g_v], tz_v, semz)
        cx.wait()
        pltpu.sync_copy(tx_v, ox_hbm.at[pl.ds(base, b)])
        cy.wait()
        pltpu.sync_copy(ty_v, oy_hbm.at[pl.ds(base, b)])
        cz.wait()
        pltpu.sync_copy(tz_v, oz_hbm.at[pl.ds(base, b)])

    return k(f_flat, vorder, vx, vy, vz)


def _graph_from_columns(c, F):
    eps = 1e-8

    def sub(p, q):
        return [p[l] - q[l] for l in range(3)]

    def dot3(p, q):
        return p[0] * q[0] + p[1] * q[1] + p[2] * q[2]

    def norm3(p):
        return jnp.sqrt(dot3(p, p))

    def unit(p):
        n = norm3(p) + eps
        return [p[l] / n for l in range(3)]

    v0, v1, v2 = c
    e1 = sub(v1, v0)
    e2 = sub(v2, v0)
    nx = e1[1] * e2[2] - e1[2] * e2[1]
    ny = e1[2] * e2[0] - e1[0] * e2[2]
    nz = e1[0] * e2[1] - e1[1] * e2[0]
    nn = jnp.sqrt(nx * nx + ny * ny + nz * nz)
    area = nn * 0.5
    inn = 1.0 / (nn + eps)

    def ang(p, q):
        return jnp.arccos(jnp.clip(dot3(unit(p), unit(q)), -1.0, 1.0))

    a0 = ang(sub(v1, v0), sub(v2, v0))
    a1 = ang(sub(v0, v1), sub(v2, v1))
    a2 = ang(sub(v0, v2), sub(v1, v2))
    colset = [
        v0[0], v0[1], v0[2], v1[0], v1[1], v1[2], v2[0], v2[1], v2[2],
        nx * inn, ny * inn, nz * inn, a0, a1, a2, area,
    ]
    return jnp.stack([col[:F] for col in colset], axis=1)


def _mlp_body(x_ref, w1_ref, b1_ref, w2_ref, b2_ref, p_ref, m_ref, o_ref):
    h = jnp.maximum(
        jnp.dot(x_ref[...], w1_ref[...], precision=_PREC) + b1_ref[...], 0.0
    )
    o = jnp.maximum(
        jnp.dot(h, w2_ref[...], precision=_PREC) + b2_ref[...], 0.0
    )
    o_ref[...] = jnp.where(m_ref[...] > 0.0, p_ref[...], o)


@functools.partial(jax.jit, static_argnames=("block_rows",))
def _mlp(x, W1, b1, W2, b2, patch, mask, block_rows=2048):
    F, Din = x.shape
    Dmid = W1.shape[1]
    Dout = W2.shape[1]
    return pl.pallas_call(
        _mlp_body,
        grid=(pl.cdiv(F, block_rows),),
        in_specs=[
            pl.BlockSpec((block_rows, Din), lambda i: (i, 0)),
            pl.BlockSpec((Din, Dmid), lambda i: (0, 0)),
            pl.BlockSpec((1, Dmid), lambda i: (0, 0)),
            pl.BlockSpec((Dmid, Dout), lambda i: (0, 0)),
            pl.BlockSpec((1, Dout), lambda i: (0, 0)),
            pl.BlockSpec((block_rows, Dout), lambda i: (i, 0)),
            pl.BlockSpec((block_rows, 1), lambda i: (i, 0)),
        ],
        out_specs=pl.BlockSpec((block_rows, Dout), lambda i: (i, 0)),
        out_shape=jax.ShapeDtypeStruct((F, Dout), jnp.float32),
    )(x, W1, b1.reshape(1, Dmid), W2, b2.reshape(1, Dout), patch,
      mask.reshape(F, 1))


def _dense_body(x_ref, m_ref, ws_ref, wn_ref, b_ref, o_ref):
    acc = jnp.dot(x_ref[...], ws_ref[...], precision=_PREC)
    acc = acc + jnp.dot(m_ref[...], wn_ref[...], precision=_PREC)
    o_ref[...] = jnp.maximum(acc + b_ref[...], 0.0)


@functools.partial(jax.jit, static_argnames=("block_rows",))
def _dense_layer(x, mean, Ws, Wn, b, block_rows=1024):
    F, Din = x.shape
    Dout = Ws.shape[1]
    return pl.pallas_call(
        _dense_body,
        grid=(pl.cdiv(F, block_rows),),
        in_specs=[
            pl.BlockSpec((block_rows, Din), lambda i: (i, 0)),
            pl.BlockSpec((block_rows, Din), lambda i: (i, 0)),
            pl.BlockSpec((Din, Dout), lambda i: (0, 0)),
            pl.BlockSpec((Din, Dout), lambda i: (0, 0)),
            pl.BlockSpec((1, Dout), lambda i: (0, 0)),
        ],
        out_specs=pl.BlockSpec((block_rows, Dout), lambda i: (i, 0)),
        out_shape=jax.ShapeDtypeStruct((F, Dout), jnp.float32),
    )(x, mean, Ws, Wn, b.reshape(1, Dout))


def _bag(x, N6, sw, cnt):
    F = x.shape[0]
    acc = sw[:, None] * x
    for t in range(6):
        idx = N6[:, t]
        valid = idx < F
        acc = acc + jnp.where(valid[:, None], x[jnp.minimum(idx, F - 1)], 0.0)
    return acc / jnp.maximum(cnt, 1.0)[:, None]


def kernel(vertices, faces, Ws1, Wn1, b1, Ws2, Wn2, b2):
    faces = faces.astype(jnp.int32)
    F = faces.shape[0]
    S = 3 * F
    Nv = vertices.shape[0]

    vorder = jnp.lexsort(
        (vertices[:, 2], vertices[:, 1], vertices[:, 0])
    ).astype(jnp.int32)
    inv = (
        jnp.zeros((Nv,), jnp.int32)
        .at[vorder]
        .set(jnp.arange(Nv, dtype=jnp.int32))
    )
    faces_flat = _pad_to(faces.reshape(-1), 256)
    f = _sc_elem_gather(inv, faces_flat)[:S].reshape(F, 3)
    fa, fb, fc = f[:, 0], f[:, 1], f[:, 2]
    lo01, hi01 = jnp.minimum(fa, fb), jnp.maximum(fa, fb)
    g0 = jnp.minimum(lo01, fc)
    g2 = jnp.maximum(hi01, fc)
    g1 = jnp.minimum(jnp.maximum(lo01, fc), hi01)
    f = jnp.stack([g0, g1, g2], axis=1)
    forder = jnp.lexsort((f[:, 2], g0 * Nv + g1)).astype(jnp.int32)
    finv = (
        jnp.zeros((F,), jnp.int32)
        .at[forder]
        .set(jnp.arange(F, dtype=jnp.int32))
    )

    slot_src = (
        forder[None, :] * 3 + jnp.arange(3, dtype=jnp.int32)[:, None]
    ).reshape(-1)
    f_flat = f.reshape(-1)
    fvals = _sc_elem_gather(_pad_to(f_flat, 256), _pad_to(slot_src, 256))
    cx, cy, cz = _sc_tri_columns(
        fvals, vorder, vertices[:, 0], vertices[:, 1], vertices[:, 2]
    )
    c = [
        [cx[j * F : (j + 1) * F], cy[j * F : (j + 1) * F], cz[j * F : (j + 1) * F]]
        for j in range(3)
    ]
    graph = _graph_from_columns(c, F)

    av = f
    bv = jnp.roll(f, -1, axis=1)
    h = (jnp.minimum(av, bv) * Nv + jnp.maximum(av, bv)).reshape(-1)
    sslot = (finv[:, None] * 3 + jnp.arange(3, dtype=jnp.int32)).reshape(-1)
    hs, order = lax.sort((h, sslot), num_keys=2)
    fs = order // 3

    same_l = jnp.concatenate([jnp.zeros((1,), jnp.bool_), hs[1:] == hs[:-1]])
    npairs = jnp.sum(same_l.astype(jnp.int32))
    f0 = fs[0]

    def fast(ops):
        graph, fs, same_l, f0, Ws1, Wn1, b1, Ws2, Wn2, b2 = ops

        (P,) = jnp.nonzero(same_l, size=_PCAP, fill_value=S)
        valid = P < S
        Pc = jnp.minimum(P, S - 1)
        idx2 = jnp.concatenate([Pc, jnp.maximum(Pc - 1, 0)])
        fs2 = _sc_elem_gather(fs, idx2)
        rP = fs2[:_PCAP]
        sP = fs2[_PCAP:]
        ends = fs2
        vmask2 = jnp.concatenate([valid, valid])
        ends_s = jnp.where(vmask2, ends, F + 1)

        hasnbr = jnp.zeros((F,), jnp.bool_).at[ends_s].set(True, mode="drop")
        (idxA,) = jnp.nonzero(hasnbr, size=_CAP, fill_value=F + 1)
        idxAc = jnp.minimum(idxA, F - 1)
        posA = jnp.full((F + 2,), _CAP, jnp.int32).at[idxA].set(
            jnp.arange(_CAP, dtype=jnp.int32), mode="drop"
        )
        posA = posA.at[F].set(_CAP).at[F + 1].set(_CAP)

        pos2 = _sc_elem_gather(posA, jnp.where(vmask2, ends, F))
        pr = pos2[:_PCAP]
        ps = pos2[_PCAP:]
        rev2 = jnp.concatenate([ps, pr])

        onesv2 = jnp.where(vmask2, 1.0, 0.0)
        adj_flat = (
            jnp.zeros(((_CAP + 1) * (_CAP + 1),), jnp.float32)
            .at[pos2 * (_CAP + 1) + rev2]
            .add(onesv2)
        )
        Adj = adj_flat.reshape(_CAP + 1, _CAP + 1)[:_CAP, :_CAP]

        nL = jnp.zeros((_CAP + 1,), jnp.float32).at[pr].add(
            jnp.where(valid, 1.0, 0.0)
        )[:_CAP]
        nR = jnp.zeros((_CAP + 1,), jnp.float32).at[ps].add(
            jnp.where(valid, 1.0, 0.0)
        )[:_CAP]
        pos0A = (idxA == f0).astype(jnp.float32)
        cntA = jnp.maximum(6.0 - nL + nR - 2.0 * pos0A, 1.0)[:, None]
        swA = (6.0 - 2.0 * nL - 2.0 * pos0A)[:, None]

        xA = graph[idxAc]
        nbr1 = jnp.dot(Adj, xA, precision=_PREC)
        mean1 = (nbr1 + swA * xA) / cntA
        hA = jax.nn.relu(xA @ Ws1 + mean1 @ Wn1 + b1)
        nbr2 = jnp.dot(Adj, hA, precision=_PREC)
        mean2 = (nbr2 + swA * hA) / cntA
        outA = jax.nn.relu(hA @ Ws2 + mean2 @ Wn2 + b2)

        patch = jnp.zeros((F, outA.shape[1]), jnp.float32).at[idxA].set(
            outA, mode="drop"
        )
        return _mlp(
            graph, Ws1 + Wn1, b1, Ws2 + Wn2, b2, patch,
            hasnbr.astype(jnp.float32),
        )

    def slow(ops):
        graph, fs, same_l, f0, Ws1, Wn1, b1, Ws2, Wn2, b2 = ops
        same_r = jnp.concatenate([same_l[1:], jnp.zeros((1,), jnp.bool_)])
        pos = jnp.arange(S)
        selfw = jnp.where(jnp.logical_and(pos > 0, ~same_l), 2.0, 0.0)
        percnt = same_l.astype(jnp.float32) + same_r.astype(jnp.float32) + selfw
        left_nb = jnp.where(same_l, jnp.roll(fs, 1), F)
        right_nb = jnp.where(same_r, jnp.roll(fs, -1), F)

        def to_slot(v):
            return jnp.zeros(S, v.dtype).at[order].set(v)

        LN = to_slot(left_nb).reshape(F, 3)
        RN = to_slot(right_nb).reshape(F, 3)
        sw = to_slot(selfw).reshape(F, 3).sum(axis=1)
        cnt = to_slot(percnt).reshape(F, 3).sum(axis=1)
        N6 = jnp.concatenate([LN, RN], axis=1).astype(jnp.int32)

        mean1 = _bag(graph, N6, sw, cnt)
        hh = _dense_layer(graph, mean1, Ws1, Wn1, b1)
        mean2 = _bag(hh, N6, sw, cnt)
        return _dense_layer(hh, mean2, Ws2, Wn2, b2)

    ops = (graph, fs, same_l, f0, Ws1, Wn1, b1, Ws2, Wn2, b2)
    return lax.cond(npairs <= min(_PCAP, _CAP // 2), fast, slow, ops)
